# Initial kernel scaffold; baseline (speedup 1.0000x reference)
import jax
import jax.numpy as jnp
from jax.experimental import pallas as pl


def kernel(indices, table):
    raise NotImplementedError("write your pallas kernel here")


# speedup vs baseline: 1.3065x; 1.3065x over previous
import functools

import jax
import jax.numpy as jnp
from jax import lax
from jax.experimental import pallas as pl
from jax.experimental.pallas import tpu as pltpu
from jax.experimental.pallas import tpu_sc as plsc

_CH = 128


def kernel(indices, table):
    B, T = indices.shape
    V, D = table.shape
    n = B * T
    info = plsc.get_sparse_core_info()
    NC, NS = info.num_cores, info.num_subcores
    NW = NC * NS
    per_w = n // NW
    k = per_w // _CH
    assert per_w * NW == n and k * _CH == per_w

    idx3 = indices.reshape(NW, k, _CH).astype(jnp.int32)
    mesh = plsc.VectorSubcoreMesh(core_axis_name="c", subcore_axis_name="s")

    @functools.partial(
        pl.kernel,
        mesh=mesh,
        out_type=jax.ShapeDtypeStruct((n, D), jnp.float32),
        scratch_types=[
            pltpu.VMEM((k, _CH), jnp.int32),
            pltpu.VMEM((_CH, D), jnp.float32),
            pltpu.SemaphoreType.DMA,
        ],
        compiler_params=pltpu.CompilerParams(use_tc_tiling_on_sc=False),
    )
    def run(idx_hbm, tab_hbm, out_hbm, idx_v, rows_v, sem):
        wid = lax.axis_index("s") * NC + lax.axis_index("c")
        base = wid * per_w
        pltpu.sync_copy(idx_hbm.at[wid], idx_v)

        def body(j, carry):
            pltpu.async_copy(tab_hbm.at[idx_v.at[j]], rows_v, sem).wait()
            pltpu.sync_copy(rows_v, out_hbm.at[pl.ds(base + j * _CH, _CH)])
            return carry

        lax.fori_loop(0, k, body, 0)

    out = run(idx3, table)
    return out.reshape(B, T, D)


# speedup vs baseline: 1.4939x; 1.1435x over previous
import functools

import jax
import jax.numpy as jnp
from jax import lax
from jax.experimental import pallas as pl
from jax.experimental.pallas import tpu as pltpu
from jax.experimental.pallas import tpu_sc as plsc

_CH = 1024


def kernel(indices, table):
    B, T = indices.shape
    V, D = table.shape
    n = B * T
    info = plsc.get_sparse_core_info()
    NC, NS = info.num_cores, info.num_subcores
    NW = NC * NS
    per_w = n // NW
    k = per_w // _CH
    assert per_w * NW == n and k * _CH == per_w

    idx3 = indices.reshape(NW, k, _CH).astype(jnp.int32)
    mesh = plsc.VectorSubcoreMesh(core_axis_name="c", subcore_axis_name="s")

    @functools.partial(
        pl.kernel,
        mesh=mesh,
        out_type=jax.ShapeDtypeStruct((n, D), jnp.float32),
        scratch_types=[
            pltpu.VMEM((k, _CH), jnp.int32),
            pltpu.VMEM((2, _CH, D), jnp.float32),
            pltpu.SemaphoreType.DMA,
        ],
        compiler_params=pltpu.CompilerParams(use_tc_tiling_on_sc=False),
    )
    def run(idx_hbm, tab_hbm, out_hbm, idx_v, buf, sem):
        wid = lax.axis_index("s") * NC + lax.axis_index("c")
        base = wid * per_w
        pltpu.sync_copy(idx_hbm.at[wid], idx_v)

        pltpu.async_copy(tab_hbm.at[idx_v.at[0]], buf.at[0], sem)

        def body(j, carry):
            p = lax.rem(j, 2)
            pltpu.make_async_copy(tab_hbm.at[idx_v.at[j]], buf.at[p], sem).wait()

            @pl.when(j + 1 < k)
            def _():
                pltpu.async_copy(tab_hbm.at[idx_v.at[j + 1]], buf.at[1 - p], sem)

            pltpu.sync_copy(buf.at[p], out_hbm.at[pl.ds(base + j * _CH, _CH)])
            return carry

        lax.fori_loop(0, k, body, 0)

    out = run(idx3, table)
    return out.reshape(B, T, D)
